# Initial kernel scaffold; baseline (speedup 1.0000x reference)
import jax
import jax.numpy as jnp
from jax.experimental import pallas as pl


def kernel(x, img_dim):
    raise NotImplementedError("write your pallas kernel here")


# speedup vs baseline: 2.0721x; 2.0721x over previous
import functools

import jax
import jax.numpy as jnp
from jax.experimental import pallas as pl
from jax.experimental.pallas import tpu as pltpu

_B = 16
_A = 3
_C = 85
_H = 76
_W = 76
_HW = _H * _W
_K = _A * _C

_ANCHORS_ALL = [[10, 13], [16, 30], [33, 23], [30, 61], [62, 45],
                [59, 119], [116, 90], [156, 198], [373, 326]]
_MASK = [0, 1, 2]


def _decode_body(params_ref, x_ref, o_ref):
    v = x_ref[0]
    y = v.T
    stride = params_ref[0]

    k = jax.lax.broadcasted_iota(jnp.int32, (_HW, _K), 1)
    c = k % _C
    a = k // _C
    hw = jax.lax.broadcasted_iota(jnp.int32, (_HW, _K), 0)
    w = (hw % _W).astype(jnp.float32)
    h = (hw // _W).astype(jnp.float32)

    sig = jax.nn.sigmoid(y)
    mesh = jnp.where(c == 0, w, h)
    xy = (sig + mesh) * stride

    aw = jnp.where(a == 0, params_ref[1], jnp.where(a == 1, params_ref[3], params_ref[5]))
    ah = jnp.where(a == 0, params_ref[2], jnp.where(a == 1, params_ref[4], params_ref[6]))
    anchor = jnp.where(c == 2, aw, ah)
    wh = jnp.exp(y) * anchor * stride

    o_ref[0] = jnp.where(c < 2, xy, jnp.where(c < 4, wh, sig))


def kernel(x, img_dim):
    x3 = x.reshape(_B, _K, _HW)
    stride = (img_dim[1] / _H).astype(jnp.float32)
    anchors = jnp.asarray(
        [_ANCHORS_ALL[i] for i in _MASK], dtype=jnp.float32).reshape(-1) / stride
    params = jnp.concatenate([stride[None], anchors, jnp.zeros((1,), jnp.float32)])

    out = pl.pallas_call(
        _decode_body,
        grid=(_B,),
        in_specs=[
            pl.BlockSpec(memory_space=pltpu.SMEM),
            pl.BlockSpec((1, _K, _HW), lambda b: (b, 0, 0)),
        ],
        out_specs=pl.BlockSpec((1, _HW, _K), lambda b: (b, 0, 0)),
        out_shape=jax.ShapeDtypeStruct((_B, _HW, _K), jnp.float32),
    )(params, x3)
    return out.reshape(_B, _A * _HW, _C)
